# Initial kernel scaffold; baseline (speedup 1.0000x reference)
import jax
import jax.numpy as jnp
from jax.experimental import pallas as pl


def kernel(x, pos, edge_index, edge_attr, batch, W_in, b_in, msg_W1, msg_b1, msg_W2, msg_b2, pos_W1, pos_b1, pos_W2, pos_b2, upd_W1, upd_b1, upd_W2, upd_b2, W_pred, b_pred):
    raise NotImplementedError("write your pallas kernel here")


# speedup vs baseline: 1.5614x; 1.5614x over previous
import functools

import jax
import jax.numpy as jnp
from jax import lax
from jax.experimental import pallas as pl
from jax.experimental.pallas import tpu as pltpu
from jax.experimental.pallas import tpu_sc as plsc

N = 50000
E = 800000
D = 64
ED = 4
G = 128
TW = 128
SW = 128
NC = 2
NS = 16
NW = NC * NS
CHUNK = 128
NCH = E // CHUNK
NP = 4
UW = 6400
WPAD = UW + CHUNK
TRASH = UW + 64
NWIN = NP * NC
VL = 16

BE = 2000
BN = 1000


def _embed_body(x_ref, pos_ref, w_ref, b_ref, t_ref):
    h = jnp.dot(x_ref[...], w_ref[...], preferred_element_type=jnp.float32)
    h = h + b_ref[...]
    t_ref[...] = jnp.concatenate(
        [h, pos_ref[...], jnp.zeros((h.shape[0], TW - D - 3), jnp.float32)],
        axis=1)


def _embed(x, pos, w_in, b_in):
    grid = (N // BN,)
    return pl.pallas_call(
        _embed_body,
        grid=grid,
        in_specs=[
            pl.BlockSpec((BN, 11), lambda i: (i, 0)),
            pl.BlockSpec((BN, 3), lambda i: (i, 0)),
            pl.BlockSpec((11, D), lambda i: (0, 0)),
            pl.BlockSpec((1, D), lambda i: (0, 0)),
        ],
        out_specs=pl.BlockSpec((BN, TW), lambda i: (i, 0)),
        out_shape=jax.ShapeDtypeStruct((N, TW), jnp.float32),
    )(x, pos, w_in, b_in.reshape(1, D))


def _gather_body(t_hbm, src_hbm, dst_hbm, outd_hbm, outs_hbm,
                 idxd_v, idxs_v, rowsd_v, rowss_v, sem):
    cid = lax.axis_index("c")
    sid = lax.axis_index("s")
    wid = sid * NC + cid
    max_k = (NCH + NW - 1) // NW

    def body(k, carry):
        c = wid + k * NW

        @pl.when(c < NCH)
        def _():
            base = c * CHUNK
            pltpu.sync_copy(dst_hbm.at[pl.ds(base, CHUNK)], idxd_v)
            pltpu.sync_copy(src_hbm.at[pl.ds(base, CHUNK)], idxs_v)
            pltpu.async_copy(t_hbm.at[idxd_v], rowsd_v, sem).wait()
            pltpu.async_copy(t_hbm.at[idxs_v], rowss_v, sem).wait()
            pltpu.sync_copy(rowsd_v, outd_hbm.at[pl.ds(base, CHUNK)])
            pltpu.sync_copy(rowss_v, outs_hbm.at[pl.ds(base, CHUNK)])
        return carry

    lax.fori_loop(0, max_k, body, None)


@functools.cache
def _make_gather_call():
    return pl.kernel(
        _gather_body,
        out_type=(
            jax.ShapeDtypeStruct((E, TW), jnp.float32),
            jax.ShapeDtypeStruct((E, TW), jnp.float32),
        ),
        mesh=plsc.VectorSubcoreMesh(core_axis_name="c", subcore_axis_name="s",
                                    num_cores=NC, num_subcores=NS),
        scratch_types=[
            pltpu.VMEM((CHUNK,), jnp.int32),
            pltpu.VMEM((CHUNK,), jnp.int32),
            pltpu.VMEM((CHUNK, TW), jnp.float32),
            pltpu.VMEM((CHUNK, TW), jnp.float32),
            pltpu.SemaphoreType.DMA,
        ],
    )


def _edge_body(tgd_ref, tgs_ref, ea_ref,
               w1_ref, b1_ref,
               w2_ref, b2_ref, pw1_ref, pb1_ref, pw2_ref, pb2_ref,
               rows_ref):
    tgd = tgd_ref[...]
    tgs = tgs_ref[...]
    h_i = tgd[:, :D]
    h_j = tgs[:, :D]
    d_vec = tgd[:, D:D + 3] - tgs[:, D:D + 3]
    dist = jnp.sqrt(jnp.sum(d_vec * d_vec, axis=1, keepdims=True) + 1e-12)
    m_in = jnp.concatenate([h_i, h_j, ea_ref[...], dist], axis=1)
    m = jnp.dot(m_in, w1_ref[...], preferred_element_type=jnp.float32)
    m = m + b1_ref[...]
    m = jnp.maximum(m, 0.0)
    m = jnp.dot(m, w2_ref[...], preferred_element_type=jnp.float32) + b2_ref[...]
    m = jnp.maximum(m, 0.0)
    sh = jnp.dot(m, pw1_ref[...], preferred_element_type=jnp.float32) + pb1_ref[...]
    sh = jnp.maximum(sh, 0.0)
    s = jnp.dot(sh, pw2_ref[...], preferred_element_type=jnp.float32) + pb2_ref[...]
    pos_msg = d_vec * s
    nrows = m.shape[0]
    rows_ref[...] = jnp.concatenate(
        [m, pos_msg, jnp.ones((nrows, 1), jnp.float32),
         jnp.zeros((nrows, SW - D - 4), jnp.float32)], axis=1)


def _edge_mlp(tgd, tgs, ea, w1, b1, w2, b2, pw1, pb1, pw2, pb2):
    grid = (E // BE,)
    full = lambda i: (0, 0)
    return pl.pallas_call(
        _edge_body,
        grid=grid,
        in_specs=[
            pl.BlockSpec((BE, TW), lambda i: (i, 0)),
            pl.BlockSpec((BE, TW), lambda i: (i, 0)),
            pl.BlockSpec((BE, ED), lambda i: (i, 0)),
            pl.BlockSpec((2 * D + ED + 1, D), full),
            pl.BlockSpec((1, D), full),
            pl.BlockSpec((D, D), full),
            pl.BlockSpec((1, D), full),
            pl.BlockSpec((D, D), full),
            pl.BlockSpec((1, D), full),
            pl.BlockSpec((D, 1), full),
            pl.BlockSpec((1, 1), full),
        ],
        out_specs=pl.BlockSpec((BE, SW), lambda i: (i, 0)),
        out_shape=jax.ShapeDtypeStruct((E, SW), jnp.float32),
    )(tgd, tgs, ea, w1, b1, w2, b2, pw1, pb1, pw2, pb2)


def _scatter_body(rows_hbm, dst_hbm, zeros_hbm, out_hbm,
                  idx_v, idx2_v, data_v, table_sh):
    cid = lax.axis_index("c")
    sid = lax.axis_index("s")
    nzch = UW // CHUNK + 1
    noch = UW // CHUNK
    max_z = (nzch + NS - 1) // NS
    max_o = (noch + NS - 1) // NS
    max_k = (NCH + NS - 1) // NS

    for p in range(NP):
        base = (p * NC + cid) * UW

        def zero_body(k, carry):
            c = sid + k * NS

            @pl.when(c < nzch)
            def _():
                pltpu.sync_copy(zeros_hbm, table_sh.at[pl.ds(c * CHUNK, CHUNK)])
            return carry

        lax.fori_loop(0, max_z, zero_body, None)
        plsc.subcore_barrier()

        def body(k, carry):
            c = sid + k * NS

            @pl.when(c < NCH)
            def _():
                eb = c * CHUNK
                pltpu.sync_copy(dst_hbm.at[pl.ds(eb, CHUNK)], idx_v)
                pltpu.sync_copy(rows_hbm.at[pl.ds(eb, CHUNK)], data_v)
                for j in range(CHUNK // VL):
                    v = idx_v[pl.ds(j * VL, VL)] - base
                    ok = (v >= 0) & (v < UW)
                    idx2_v[pl.ds(j * VL, VL)] = jnp.where(ok, v, TRASH)
                pltpu.sync_copy(data_v, table_sh.at[idx2_v], add=True)
            return carry

        lax.fori_loop(0, max_k, body, None)
        plsc.subcore_barrier()

        def out_body(k, carry):
            c = sid + k * NS

            @pl.when(c < noch)
            def _():
                pltpu.sync_copy(table_sh.at[pl.ds(c * CHUNK, CHUNK)],
                                out_hbm.at[pl.ds(base + c * CHUNK, CHUNK)])
            return carry

        lax.fori_loop(0, max_o, out_body, None)
        plsc.subcore_barrier()


@functools.cache
def _make_scatter_call():
    return pl.kernel(
        _scatter_body,
        out_type=jax.ShapeDtypeStruct((NWIN * UW, SW), jnp.float32),
        mesh=plsc.VectorSubcoreMesh(core_axis_name="c", subcore_axis_name="s",
                                    num_cores=NC, num_subcores=NS),
        scratch_types=[
            pltpu.VMEM((CHUNK,), jnp.int32),
            pltpu.VMEM((CHUNK,), jnp.int32),
            pltpu.VMEM((CHUNK, SW), jnp.float32),
            pltpu.VMEM_SHARED((WPAD, SW), jnp.float32),
        ],
    )


def _node_body(t_ref, sc_ref, w1_ref, b1_ref, w2_ref, b2_ref, tn_ref):
    t = t_ref[...]
    h = t[:, :D]
    p = t[:, D:D + 3]
    sc = sc_ref[...]
    aggr = sc[:, :D]
    pos_sum = sc[:, D:D + 3]
    deg = jnp.maximum(sc[:, D + 3:D + 4], 1.0)
    u_in = jnp.concatenate([h, aggr], axis=1)
    u = jnp.dot(u_in, w1_ref[...], preferred_element_type=jnp.float32) + b1_ref[...]
    u = jnp.maximum(u, 0.0)
    u = jnp.dot(u, w2_ref[...], preferred_element_type=jnp.float32) + b2_ref[...]
    u = jnp.maximum(u, 0.0)
    h_new = h + u
    p_new = p + pos_sum / deg
    tn_ref[...] = jnp.concatenate(
        [h_new, p_new, jnp.zeros((h.shape[0], TW - D - 3), jnp.float32)],
        axis=1)


def _node_mlp(t, scat, w1, b1, w2, b2):
    grid = (N // BN,)
    full = lambda i: (0, 0)
    return pl.pallas_call(
        _node_body,
        grid=grid,
        in_specs=[
            pl.BlockSpec((BN, TW), lambda i: (i, 0)),
            pl.BlockSpec((BN, SW), lambda i: (i, 0)),
            pl.BlockSpec((2 * D, D), full),
            pl.BlockSpec((1, D), full),
            pl.BlockSpec((D, D), full),
            pl.BlockSpec((1, D), full),
        ],
        out_specs=pl.BlockSpec((BN, TW), lambda i: (i, 0)),
        out_shape=jax.ShapeDtypeStruct((N, TW), jnp.float32),
    )(t, scat, w1, b1, w2, b2)


def _pool_body(t_ref, batch_ref, wp_ref, bp_ref, out_ref, acc, cnt):
    i = pl.program_id(0)

    @pl.when(i == 0)
    def _():
        acc[...] = jnp.zeros_like(acc)
        cnt[...] = jnp.zeros_like(cnt)

    h = t_ref[...][:, :D]
    bb = batch_ref[0]
    onehot = (lax.broadcasted_iota(jnp.int32, (G, BN), 0) == bb).astype(
        jnp.float32)
    acc[...] += jnp.dot(onehot, h, preferred_element_type=jnp.float32)
    cnt[...] += jnp.sum(onehot, axis=1, keepdims=True)

    @pl.when(i == pl.num_programs(0) - 1)
    def _():
        hg = acc[...] / jnp.maximum(cnt[...], 1.0)
        out_ref[...] = jnp.dot(hg, wp_ref[...],
                               preferred_element_type=jnp.float32) + bp_ref[...]


def _pool(t, batch2d, w_pred, b_pred):
    grid = (N // BN,)
    full = lambda i: (0, 0)
    return pl.pallas_call(
        _pool_body,
        grid=grid,
        in_specs=[
            pl.BlockSpec((BN, TW), lambda i: (i, 0)),
            pl.BlockSpec((1, 1, BN), lambda i: (i, 0, 0)),
            pl.BlockSpec((D, 1), full),
            pl.BlockSpec((1, 1), full),
        ],
        out_specs=pl.BlockSpec((G, 1), full),
        out_shape=jax.ShapeDtypeStruct((G, 1), jnp.float32),
        scratch_shapes=[
            pltpu.VMEM((G, D), jnp.float32),
            pltpu.VMEM((G, 1), jnp.float32),
        ],
    )(t, batch2d.reshape(N // BN, 1, BN), w_pred, b_pred.reshape(1, 1))


def kernel(x, pos, edge_index, edge_attr, batch, W_in, b_in,
           msg_W1, msg_b1, msg_W2, msg_b2,
           pos_W1, pos_b1, pos_W2, pos_b2,
           upd_W1, upd_b1, upd_W2, upd_b2,
           W_pred, b_pred):
    L = msg_W1.shape[0]
    src = edge_index[0]
    dst = edge_index[1]
    t = _embed(x, pos, W_in, b_in)
    zeros_chunk = jnp.zeros((CHUNK, SW), jnp.float32)
    for l in range(L):
        tgd, tgs = _make_gather_call()(t, src, dst)
        rows = _edge_mlp(
            tgd, tgs, edge_attr,
            msg_W1[l], msg_b1[l].reshape(1, D),
            msg_W2[l], msg_b2[l].reshape(1, D),
            pos_W1[l], pos_b1[l].reshape(1, D),
            pos_W2[l], pos_b2[l].reshape(1, 1))
        scat = _make_scatter_call()(rows, dst, zeros_chunk)
        t = _node_mlp(t, scat, upd_W1[l], upd_b1[l].reshape(1, D),
                      upd_W2[l], upd_b2[l].reshape(1, D))
    out = _pool(t, batch.reshape(N // BN, BN), W_pred, b_pred)
    return out.reshape(-1)
